# initial kernel scaffold (unmeasured)
import jax
import jax.numpy as jnp
from jax import lax
from jax.experimental import pallas as pl
from jax.experimental.pallas import tpu as pltpu

B = 8
SQ = 8
H = 16
D = 128
SKV = 1024
N_Z = 4
SCALE = D ** -0.5


def _attn_body(q_ref, k_ref, v_ref, o_ref, l_ref):
    q = q_ref[0, :, 0, :]
    k = k_ref[0, :, 0, :]
    v = v_ref[0, :, 0, :]
    s = lax.dot_general(q, k, (((1,), (1,)), ((), ())),
                        preferred_element_type=jnp.float32) * SCALE
    p = jnp.exp(s)
    l_ref[0, 0, :] = jnp.sum(p, axis=1)
    o_ref[0, :, 0, :] = lax.dot_general(p, v, (((1,), (0,)), ((), ())),
                                        preferred_element_type=jnp.float32)


def _allreduce_body(o_ref, l_ref, out_ref, lacc_ref, obuf, lbuf,
                    o_send, o_recv, l_send, l_recv):
    x = lax.axis_index("x")
    y = lax.axis_index("y")
    z = lax.axis_index("z")
    left = (z - 1) % N_Z
    right = (z + 1) % N_Z

    barrier = pltpu.get_barrier_semaphore()
    for nbr in (left, right):
        pl.semaphore_signal(barrier, inc=1, device_id=(x, y, nbr),
                            device_id_type=pl.DeviceIdType.MESH)
    pl.semaphore_wait(barrier, 2)

    out_ref[...] = o_ref[...]
    lacc_ref[...] = l_ref[...]
    obuf[0] = o_ref[...]
    lbuf[0] = l_ref[...]

    for h in range(N_Z - 1):
        o_rdma = pltpu.make_async_remote_copy(
            src_ref=obuf.at[h], dst_ref=obuf.at[h + 1],
            send_sem=o_send.at[h], recv_sem=o_recv.at[h],
            device_id=(x, y, right), device_id_type=pl.DeviceIdType.MESH)
        l_rdma = pltpu.make_async_remote_copy(
            src_ref=lbuf.at[h], dst_ref=lbuf.at[h + 1],
            send_sem=l_send.at[h], recv_sem=l_recv.at[h],
            device_id=(x, y, right), device_id_type=pl.DeviceIdType.MESH)
        o_rdma.start()
        l_rdma.start()
        o_rdma.wait()
        l_rdma.wait()
        out_ref[...] = out_ref[...] + obuf[h + 1]
        lacc_ref[...] = lacc_ref[...] + lbuf[h + 1]

    for hh in range(H):
        out_ref[:, :, hh, :] = (
            out_ref[:, :, hh, :] / lacc_ref[:, hh, :][:, :, None]
        )


def kernel(Q, K, V):
    o_un, l_un = pl.pallas_call(
        _attn_body,
        grid=(B, H),
        in_specs=[
            pl.BlockSpec((1, SQ, 1, D), lambda b, h: (b, 0, h, 0)),
            pl.BlockSpec((1, SKV, 1, D), lambda b, h: (b, 0, h, 0)),
            pl.BlockSpec((1, SKV, 1, D), lambda b, h: (b, 0, h, 0)),
        ],
        out_specs=[
            pl.BlockSpec((1, SQ, 1, D), lambda b, h: (b, 0, h, 0)),
            pl.BlockSpec((1, 1, SQ), lambda b, h: (b, h, 0)),
        ],
        out_shape=[
            jax.ShapeDtypeStruct((B, SQ, H, D), jnp.float32),
            jax.ShapeDtypeStruct((B, H, SQ), jnp.float32),
        ],
    )(Q, K, V)

    return pl.pallas_call(
        _allreduce_body,
        out_shape=jax.ShapeDtypeStruct((B, SQ, H, D), jnp.float32),
        in_specs=[pl.BlockSpec(memory_space=pltpu.VMEM),
                  pl.BlockSpec(memory_space=pltpu.VMEM)],
        out_specs=pl.BlockSpec(memory_space=pltpu.VMEM),
        scratch_shapes=[
            pltpu.VMEM((B, H, SQ), jnp.float32),
            pltpu.VMEM((N_Z, B, SQ, H, D), jnp.float32),
            pltpu.VMEM((N_Z, B, H, SQ), jnp.float32),
            pltpu.SemaphoreType.DMA((N_Z - 1,)),
            pltpu.SemaphoreType.DMA((N_Z - 1,)),
            pltpu.SemaphoreType.DMA((N_Z - 1,)),
            pltpu.SemaphoreType.DMA((N_Z - 1,)),
        ],
        compiler_params=pltpu.CompilerParams(collective_id=0),
    )(o_un, l_un)


# baseline (device time: 221552 ns/iter reference)
import jax
import jax.numpy as jnp
from jax import lax
from jax.experimental import pallas as pl
from jax.experimental.pallas import tpu as pltpu

B = 8
SQ = 8
H = 16
D = 128
HD = H * D
SKV = 1024
KT = 512
NK = SKV // KT
N_Z = 4
SCALE = D ** -0.5


def _attn_body(q_ref, k_ref, v_ref, o_ref, l_ref):
    nk = pl.program_id(1)

    @pl.when(nk == 0)
    def _():
        o_ref[...] = jnp.zeros_like(o_ref)
        l_ref[...] = jnp.zeros_like(l_ref)

    for hh in range(H):
        sl = slice(hh * D, (hh + 1) * D)
        q2 = q_ref[0][:, sl]
        k2 = k_ref[0][:, sl]
        v2 = v_ref[0][:, sl]
        s = lax.dot_general(q2, k2, (((1,), (1,)), ((), ())),
                            preferred_element_type=jnp.float32) * SCALE
        p = jnp.exp(s)
        l_ref[0, hh, :] = l_ref[0, hh, :] + jnp.sum(p, axis=1)
        o_ref[0, :, sl] = o_ref[0, :, sl] + lax.dot_general(
            p, v2, (((1,), (0,)), ((), ())),
            preferred_element_type=jnp.float32)


def _allreduce_body(o_ref, l_ref, out_ref, lacc_ref, obuf, lbuf,
                    o_send, o_recv, l_send, l_recv):
    x = lax.axis_index("x")
    y = lax.axis_index("y")
    z = lax.axis_index("z")
    left = (z - 1) % N_Z
    right = (z + 1) % N_Z

    barrier = pltpu.get_barrier_semaphore()
    for nbr in (left, right):
        pl.semaphore_signal(barrier, inc=1, device_id=(x, y, nbr),
                            device_id_type=pl.DeviceIdType.MESH)
    pl.semaphore_wait(barrier, 2)

    out_ref[...] = o_ref[...]
    lacc_ref[...] = l_ref[...]
    obuf[0] = o_ref[...]
    lbuf[0] = l_ref[...]

    for h in range(N_Z - 1):
        o_rdma = pltpu.make_async_remote_copy(
            src_ref=obuf.at[h], dst_ref=obuf.at[h + 1],
            send_sem=o_send.at[h], recv_sem=o_recv.at[h],
            device_id=(x, y, right), device_id_type=pl.DeviceIdType.MESH)
        l_rdma = pltpu.make_async_remote_copy(
            src_ref=lbuf.at[h], dst_ref=lbuf.at[h + 1],
            send_sem=l_send.at[h], recv_sem=l_recv.at[h],
            device_id=(x, y, right), device_id_type=pl.DeviceIdType.MESH)
        o_rdma.start()
        l_rdma.start()
        o_rdma.wait()
        l_rdma.wait()
        out_ref[...] = out_ref[...] + obuf[h + 1]
        lacc_ref[...] = lacc_ref[...] + lbuf[h + 1]

    for hh in range(H):
        sl = slice(hh * D, (hh + 1) * D)
        out_ref[:, :, sl] = (
            out_ref[:, :, sl] / lacc_ref[:, hh, :][:, :, None]
        )


def kernel(Q, K, V):
    q = Q.reshape(B, SQ, HD)
    k = K.reshape(B, SKV, HD)
    v = V.reshape(B, SKV, HD)

    o_un, l_un = pl.pallas_call(
        _attn_body,
        grid=(B, NK),
        in_specs=[
            pl.BlockSpec((1, SQ, HD), lambda b, nk: (b, 0, 0)),
            pl.BlockSpec((1, KT, HD), lambda b, nk: (b, nk, 0)),
            pl.BlockSpec((1, KT, HD), lambda b, nk: (b, nk, 0)),
        ],
        out_specs=[
            pl.BlockSpec((1, SQ, HD), lambda b, nk: (b, 0, 0)),
            pl.BlockSpec((1, H, SQ), lambda b, nk: (b, 0, 0)),
        ],
        out_shape=[
            jax.ShapeDtypeStruct((B, SQ, HD), jnp.float32),
            jax.ShapeDtypeStruct((B, H, SQ), jnp.float32),
        ],
    )(q, k, v)

    out = pl.pallas_call(
        _allreduce_body,
        out_shape=jax.ShapeDtypeStruct((B, SQ, HD), jnp.float32),
        in_specs=[pl.BlockSpec(memory_space=pltpu.VMEM),
                  pl.BlockSpec(memory_space=pltpu.VMEM)],
        out_specs=pl.BlockSpec(memory_space=pltpu.VMEM),
        scratch_shapes=[
            pltpu.VMEM((B, H, SQ), jnp.float32),
            pltpu.VMEM((N_Z, B, SQ, HD), jnp.float32),
            pltpu.VMEM((N_Z, B, H, SQ), jnp.float32),
            pltpu.SemaphoreType.DMA((N_Z - 1,)),
            pltpu.SemaphoreType.DMA((N_Z - 1,)),
            pltpu.SemaphoreType.DMA((N_Z - 1,)),
            pltpu.SemaphoreType.DMA((N_Z - 1,)),
        ],
        compiler_params=pltpu.CompilerParams(collective_id=0),
    )(o_un, l_un)
    return out.reshape(B, SQ, H, D)


# device time: 215881 ns/iter; 1.0263x vs baseline; 1.0263x over previous
import jax
import jax.numpy as jnp
from jax import lax
from jax.experimental import pallas as pl
from jax.experimental.pallas import tpu as pltpu

B = 8
SQ = 8
H = 16
D = 128
HD = H * D
SKV = 1024
KT = 1024
NK = SKV // KT
N_Z = 4
SCALE = D ** -0.5


def _attn_body(q_ref, k_ref, v_ref, o_ref, l_ref):
    nk = pl.program_id(1)

    @pl.when(nk == 0)
    def _():
        o_ref[...] = jnp.zeros_like(o_ref)
        l_ref[...] = jnp.zeros_like(l_ref)

    for hh in range(H):
        sl = slice(hh * D, (hh + 1) * D)
        q2 = q_ref[0][:, sl]
        k2 = k_ref[0][:, sl]
        v2 = v_ref[0][:, sl]
        s = lax.dot_general(q2, k2, (((1,), (1,)), ((), ())),
                            preferred_element_type=jnp.float32) * SCALE
        p = jnp.exp(s)
        l_ref[0, hh, :] = l_ref[0, hh, :] + jnp.sum(p, axis=1)
        o_ref[0, :, sl] = o_ref[0, :, sl] + lax.dot_general(
            p, v2, (((1,), (0,)), ((), ())),
            preferred_element_type=jnp.float32)


def _allreduce_body(o_ref, l_ref, out_ref, lacc_ref, obuf, lbuf,
                    o_send, o_recv, l_send, l_recv):
    x = lax.axis_index("x")
    y = lax.axis_index("y")
    z = lax.axis_index("z")
    left = (z - 1) % N_Z
    right = (z + 1) % N_Z

    barrier = pltpu.get_barrier_semaphore()
    for nbr in (left, right):
        pl.semaphore_signal(barrier, inc=1, device_id=(x, y, nbr),
                            device_id_type=pl.DeviceIdType.MESH)
    pl.semaphore_wait(barrier, 2)

    out_ref[...] = o_ref[...]
    lacc_ref[...] = l_ref[...]
    obuf[0] = o_ref[...]
    lbuf[0] = l_ref[...]

    for h in range(N_Z - 1):
        o_rdma = pltpu.make_async_remote_copy(
            src_ref=obuf.at[h], dst_ref=obuf.at[h + 1],
            send_sem=o_send.at[h], recv_sem=o_recv.at[h],
            device_id=(x, y, right), device_id_type=pl.DeviceIdType.MESH)
        l_rdma = pltpu.make_async_remote_copy(
            src_ref=lbuf.at[h], dst_ref=lbuf.at[h + 1],
            send_sem=l_send.at[h], recv_sem=l_recv.at[h],
            device_id=(x, y, right), device_id_type=pl.DeviceIdType.MESH)
        o_rdma.start()
        l_rdma.start()
        o_rdma.wait()
        l_rdma.wait()
        out_ref[...] = out_ref[...] + obuf[h + 1]
        lacc_ref[...] = lacc_ref[...] + lbuf[h + 1]

    for hh in range(H):
        sl = slice(hh * D, (hh + 1) * D)
        out_ref[:, :, sl] = (
            out_ref[:, :, sl] / lacc_ref[:, hh, :][:, :, None]
        )


def kernel(Q, K, V):
    q = Q.reshape(B, SQ, HD)
    k = K.reshape(B, SKV, HD)
    v = V.reshape(B, SKV, HD)

    o_un, l_un = pl.pallas_call(
        _attn_body,
        grid=(B, NK),
        in_specs=[
            pl.BlockSpec((1, SQ, HD), lambda b, nk: (b, 0, 0)),
            pl.BlockSpec((1, KT, HD), lambda b, nk: (b, nk, 0)),
            pl.BlockSpec((1, KT, HD), lambda b, nk: (b, nk, 0)),
        ],
        out_specs=[
            pl.BlockSpec((1, SQ, HD), lambda b, nk: (b, 0, 0)),
            pl.BlockSpec((1, H, SQ), lambda b, nk: (b, 0, 0)),
        ],
        out_shape=[
            jax.ShapeDtypeStruct((B, SQ, HD), jnp.float32),
            jax.ShapeDtypeStruct((B, H, SQ), jnp.float32),
        ],
    )(q, k, v)

    out = pl.pallas_call(
        _allreduce_body,
        out_shape=jax.ShapeDtypeStruct((B, SQ, HD), jnp.float32),
        in_specs=[pl.BlockSpec(memory_space=pltpu.VMEM),
                  pl.BlockSpec(memory_space=pltpu.VMEM)],
        out_specs=pl.BlockSpec(memory_space=pltpu.VMEM),
        scratch_shapes=[
            pltpu.VMEM((B, H, SQ), jnp.float32),
            pltpu.VMEM((N_Z, B, SQ, HD), jnp.float32),
            pltpu.VMEM((N_Z, B, H, SQ), jnp.float32),
            pltpu.SemaphoreType.DMA((N_Z - 1,)),
            pltpu.SemaphoreType.DMA((N_Z - 1,)),
            pltpu.SemaphoreType.DMA((N_Z - 1,)),
            pltpu.SemaphoreType.DMA((N_Z - 1,)),
        ],
        compiler_params=pltpu.CompilerParams(collective_id=0),
    )(o_un, l_un)
    return out.reshape(B, SQ, H, D)
